# baseline (speedup 1.0000x reference)
import functools

import jax
import jax.numpy as jnp
from jax import lax
from jax.experimental import pallas as pl
from jax.experimental.pallas import tpu as pltpu
from jax.experimental.pallas import tpu_sc as plsc

_EPS = 0.001
_CLIP_LO = 1e-07
_CLIP_HI = 10000000.0

_ROWS_PER_BLOCK = 9216
_CHUNK_ROWS = 64

_SC_WORKERS = 32
_SC_CHUNK_ROWS = 32
_SC_ROWS = 18432


def _make_tc_kernel(n_steps, rows_per_block, chunk_rows):
    n_chunks = rows_per_block // chunk_rows

    def _loss_block_kernel(p_ref, t_ref, s0_ref, s1_ref, n0_ref,
                           acc0_ref, acc1_ref, accn_ref):
        @pl.when(pl.program_id(0) == 0)
        def _init():
            acc0_ref[...] = jnp.zeros_like(acc0_ref)
            acc1_ref[...] = jnp.zeros_like(acc1_ref)
            accn_ref[...] = jnp.zeros_like(accn_ref)

        w = p_ref.shape[-1]
        sub = chunk_rows // 8

        def body(i, carry):
            a0, a1, an = carry
            off = i * chunk_rows
            p = p_ref[pl.ds(off, chunk_rows), :]
            t = t_ref[pl.ds(off, chunk_rows), :]
            e = p - t
            s = jnp.minimum(jnp.maximum(e * e, _CLIP_LO), _CLIP_HI)
            m0f = jnp.where(t > _EPS, 1.0, 0.0)
            s0c = s * m0f
            s1c = s - s0c
            a0 = a0 + jnp.sum(s0c.reshape(sub, 8, w), axis=0)
            a1 = a1 + jnp.sum(s1c.reshape(sub, 8, w), axis=0)
            an = an + jnp.sum(m0f.reshape(sub, 8, w), axis=0)
            return a0, a1, an

        zeros = jnp.zeros((8, w), jnp.float32)
        a0, a1, an = lax.fori_loop(0, n_chunks, body, (zeros, zeros, zeros))
        acc0_ref[...] += a0
        acc1_ref[...] += a1
        accn_ref[...] += an

        @pl.when(pl.program_id(0) == n_steps - 1)
        def _finish():
            s0_ref[...] = jnp.sum(acc0_ref[...]).reshape(1, 1, 1)
            s1_ref[...] = jnp.sum(acc1_ref[...]).reshape(1, 1, 1)
            n0_ref[...] = jnp.sum(accn_ref[...].astype(jnp.int32)).reshape(1, 1, 1)

    return _loss_block_kernel


def _tc_partial_sums(p2, t2, rows_tc):
    cols = p2.shape[1]
    n_steps = rows_tc // _ROWS_PER_BLOCK

    in_spec = pl.BlockSpec((_ROWS_PER_BLOCK, cols), lambda i: (i, 0))
    out_spec = pl.BlockSpec((1, 1, 1), lambda i: (0, 0, 0))

    return pl.pallas_call(
        _make_tc_kernel(n_steps, _ROWS_PER_BLOCK, _CHUNK_ROWS),
        grid=(n_steps,),
        in_specs=[in_spec, in_spec],
        out_specs=[out_spec, out_spec, out_spec],
        out_shape=[
            jax.ShapeDtypeStruct((1, 1, 1), jnp.float32),
            jax.ShapeDtypeStruct((1, 1, 1), jnp.float32),
            jax.ShapeDtypeStruct((1, 1, 1), jnp.int32),
        ],
        scratch_shapes=[
            pltpu.VMEM((8, cols), jnp.float32),
            pltpu.VMEM((8, cols), jnp.float32),
            pltpu.VMEM((8, cols), jnp.float32),
        ],
        compiler_params=pltpu.CompilerParams(
            dimension_semantics=("arbitrary",),
        ),
    )(p2, t2)


def _make_sc_kernel(cols, sc_row_base, rows_per_worker):
    n_chunks = rows_per_worker // _SC_CHUNK_ROWS
    assert n_chunks % 2 == 0
    vecs_per_row = cols // 16
    mesh = plsc.VectorSubcoreMesh(core_axis_name="c", subcore_axis_name="s")

    @functools.partial(
        pl.kernel,
        mesh=mesh,
        out_type=jax.ShapeDtypeStruct((_SC_WORKERS, 48), jnp.float32),
        scratch_types=[
            pltpu.VMEM((_SC_CHUNK_ROWS, cols), jnp.float32),
            pltpu.VMEM((_SC_CHUNK_ROWS, cols), jnp.float32),
            pltpu.VMEM((_SC_CHUNK_ROWS, cols), jnp.float32),
            pltpu.VMEM((_SC_CHUNK_ROWS, cols), jnp.float32),
            pltpu.VMEM((48,), jnp.float32),
            pltpu.SemaphoreType.DMA,
            pltpu.SemaphoreType.DMA,
            pltpu.SemaphoreType.DMA,
            pltpu.SemaphoreType.DMA,
        ],
    )
    def _sc_loss(p_hbm, t_hbm, out_hbm,
                 p_buf0, p_buf1, t_buf0, t_buf1, res_v,
                 sem_p0, sem_p1, sem_t0, sem_t1):
        n_cores = mesh.num_cores
        wid = lax.axis_index("s") * n_cores + lax.axis_index("c")
        base_row = sc_row_base + wid * rows_per_worker

        p_bufs = (p_buf0, p_buf1)
        t_bufs = (t_buf0, t_buf1)
        sems_p = (sem_p0, sem_p1)
        sems_t = (sem_t0, sem_t1)

        def chunk_rows_slice(idx):
            return pl.ds(base_row + idx * _SC_CHUNK_ROWS, _SC_CHUNK_ROWS)

        pltpu.async_copy(p_hbm.at[chunk_rows_slice(0)], p_bufs[0], sems_p[0])
        pltpu.async_copy(t_hbm.at[chunk_rows_slice(0)], t_bufs[0], sems_t[0])
        pltpu.async_copy(p_hbm.at[chunk_rows_slice(1)], p_bufs[1], sems_p[1])
        pltpu.async_copy(t_hbm.at[chunk_rows_slice(1)], t_bufs[1], sems_t[1])

        def chunk_sums(pb, tb, carry):
            def row_body(r, rcarry):
                a0, a1, an = rcarry
                for j in range(vecs_per_row):
                    p = pb[r, pl.ds(16 * j, 16)]
                    t = tb[r, pl.ds(16 * j, 16)]
                    e = p - t
                    s = jnp.minimum(jnp.maximum(e * e, _CLIP_LO), _CLIP_HI)
                    m0f = jnp.where(t > _EPS, 1.0, 0.0)
                    s0c = s * m0f
                    a0 = a0 + s0c
                    a1 = a1 + (s - s0c)
                    an = an + m0f
                return a0, a1, an

            return lax.fori_loop(0, _SC_CHUNK_ROWS, row_body, carry)

        def pair_body(g, carry):
            for b in range(2):
                idx = 2 * g + b
                pltpu.make_async_copy(
                    p_hbm.at[chunk_rows_slice(idx)], p_bufs[b], sems_p[b]).wait()
                pltpu.make_async_copy(
                    t_hbm.at[chunk_rows_slice(idx)], t_bufs[b], sems_t[b]).wait()
                carry = chunk_sums(p_bufs[b], t_bufs[b], carry)

                @pl.when(idx + 2 < n_chunks)
                def _prefetch():
                    pltpu.async_copy(
                        p_hbm.at[chunk_rows_slice(idx + 2)], p_bufs[b], sems_p[b])
                    pltpu.async_copy(
                        t_hbm.at[chunk_rows_slice(idx + 2)], t_bufs[b], sems_t[b])
            return carry

        zeros = jnp.zeros((16,), jnp.float32)
        a0, a1, an = lax.fori_loop(0, n_chunks // 2, pair_body,
                                   (zeros, zeros, zeros))

        res_v[pl.ds(0, 16)] = a0
        res_v[pl.ds(16, 16)] = a1
        res_v[pl.ds(32, 16)] = an
        pltpu.sync_copy(res_v, out_hbm.at[wid])

    return _sc_loss


def kernel(pred, truth):
    n_total = pred.size
    p2 = pred.reshape(-1, pred.shape[-1])
    t2 = truth.reshape(-1, truth.shape[-1])
    rows, cols = p2.shape

    rows_tc = rows - _SC_ROWS
    rows_per_worker = _SC_ROWS // _SC_WORKERS

    s0_tc, s1_tc, n0_tc = _tc_partial_sums(p2, t2, rows_tc)
    sc_parts = _make_sc_kernel(cols, rows_tc, rows_per_worker)(p2, t2)

    sc_parts = sc_parts.reshape(_SC_WORKERS, 3, 16)
    s0 = s0_tc[0, 0, 0] + jnp.sum(sc_parts[:, 0, :])
    s1 = s1_tc[0, 0, 0] + jnp.sum(sc_parts[:, 1, :])
    n0_sc = jnp.sum(sc_parts[:, 2, :].astype(jnp.int32))
    n0 = (n0_tc[0, 0, 0] + n0_sc).astype(jnp.float32)
    n1 = jnp.float32(n_total) - n0
    mean1 = s1 / jnp.maximum(n1, 1.0)
    mean0 = jnp.where(n0 > 0, s0 / jnp.maximum(n0, 1.0), mean1)
    return mean0 + mean1


# speedup vs baseline: 1.0004x; 1.0004x over previous
import functools

import jax
import jax.numpy as jnp
from jax import lax
from jax.experimental import pallas as pl
from jax.experimental.pallas import tpu as pltpu
from jax.experimental.pallas import tpu_sc as plsc

_EPS = 0.001
_CLIP_LO = 1e-07
_CLIP_HI = 10000000.0

_ROWS_PER_BLOCK = 9216
_CHUNK_ROWS = 64

_SC_WORKERS = 32
_SC_CHUNK_ROWS = 32
_SC_ROWS = 18432


def _make_tc_kernel(n_steps, rows_per_block, chunk_rows):
    n_chunks = rows_per_block // chunk_rows

    def _loss_block_kernel(p_ref, t_ref, s0_ref, s1_ref, n0_ref,
                           acc0_ref, acc1_ref, accn_ref):
        @pl.when(pl.program_id(0) == 0)
        def _init():
            acc0_ref[...] = jnp.zeros_like(acc0_ref)
            acc1_ref[...] = jnp.zeros_like(acc1_ref)
            accn_ref[...] = jnp.zeros_like(accn_ref)

        w = p_ref.shape[-1]
        sub = chunk_rows // 8

        def body(i, carry):
            a0, a1, an = carry
            off = i * chunk_rows
            p = p_ref[pl.ds(off, chunk_rows), :]
            t = t_ref[pl.ds(off, chunk_rows), :]
            e = p - t
            s = jnp.minimum(jnp.maximum(e * e, _CLIP_LO), _CLIP_HI)
            m0f = jnp.where(t > _EPS, 1.0, 0.0)
            s0c = s * m0f
            s1c = s - s0c
            a0 = a0 + jnp.sum(s0c.reshape(sub, 8, w), axis=0)
            a1 = a1 + jnp.sum(s1c.reshape(sub, 8, w), axis=0)
            an = an + jnp.sum(m0f.reshape(sub, 8, w), axis=0)
            return a0, a1, an

        zeros = jnp.zeros((8, w), jnp.float32)
        a0, a1, an = lax.fori_loop(0, n_chunks, body, (zeros, zeros, zeros))
        acc0_ref[...] += a0
        acc1_ref[...] += a1
        accn_ref[...] += an

        @pl.when(pl.program_id(0) == n_steps - 1)
        def _finish():
            s0_ref[...] = jnp.sum(acc0_ref[...]).reshape(1, 1, 1)
            s1_ref[...] = jnp.sum(acc1_ref[...]).reshape(1, 1, 1)
            n0_ref[...] = jnp.sum(accn_ref[...].astype(jnp.int32)).reshape(1, 1, 1)

    return _loss_block_kernel


def _tc_partial_sums(p2, t2, rows_tc):
    cols = p2.shape[1]
    n_steps = rows_tc // _ROWS_PER_BLOCK

    in_spec = pl.BlockSpec((_ROWS_PER_BLOCK, cols), lambda i: (i, 0))
    out_spec = pl.BlockSpec((1, 1, 1), lambda i: (0, 0, 0))

    return pl.pallas_call(
        _make_tc_kernel(n_steps, _ROWS_PER_BLOCK, _CHUNK_ROWS),
        grid=(n_steps,),
        in_specs=[in_spec, in_spec],
        out_specs=[out_spec, out_spec, out_spec],
        out_shape=[
            jax.ShapeDtypeStruct((1, 1, 1), jnp.float32),
            jax.ShapeDtypeStruct((1, 1, 1), jnp.float32),
            jax.ShapeDtypeStruct((1, 1, 1), jnp.int32),
        ],
        scratch_shapes=[
            pltpu.VMEM((8, cols), jnp.float32),
            pltpu.VMEM((8, cols), jnp.float32),
            pltpu.VMEM((8, cols), jnp.float32),
        ],
        compiler_params=pltpu.CompilerParams(
            dimension_semantics=("arbitrary",),
        ),
    )(p2, t2)


def _make_sc_kernel(cols, sc_row_base, rows_per_worker):
    n_chunks = rows_per_worker // _SC_CHUNK_ROWS
    assert n_chunks % 2 == 0
    vecs_per_row = cols // 16
    mesh = plsc.VectorSubcoreMesh(core_axis_name="c", subcore_axis_name="s")

    @functools.partial(
        pl.kernel,
        mesh=mesh,
        out_type=jax.ShapeDtypeStruct((_SC_WORKERS, 48), jnp.float32),
        scratch_types=[
            pltpu.VMEM((_SC_CHUNK_ROWS, cols), jnp.float32),
            pltpu.VMEM((_SC_CHUNK_ROWS, cols), jnp.float32),
            pltpu.VMEM((_SC_CHUNK_ROWS, cols), jnp.float32),
            pltpu.VMEM((_SC_CHUNK_ROWS, cols), jnp.float32),
            pltpu.VMEM((48,), jnp.float32),
            pltpu.SemaphoreType.DMA,
            pltpu.SemaphoreType.DMA,
            pltpu.SemaphoreType.DMA,
            pltpu.SemaphoreType.DMA,
        ],
    )
    def _sc_loss(p_hbm, t_hbm, out_hbm,
                 p_buf0, p_buf1, t_buf0, t_buf1, res_v,
                 sem_p0, sem_p1, sem_t0, sem_t1):
        n_cores = mesh.num_cores
        wid = lax.axis_index("s") * n_cores + lax.axis_index("c")
        base_row = sc_row_base + wid * rows_per_worker

        p_bufs = (p_buf0, p_buf1)
        t_bufs = (t_buf0, t_buf1)
        sems_p = (sem_p0, sem_p1)
        sems_t = (sem_t0, sem_t1)

        def chunk_rows_slice(idx):
            return pl.ds(base_row + idx * _SC_CHUNK_ROWS, _SC_CHUNK_ROWS)

        pltpu.async_copy(p_hbm.at[chunk_rows_slice(0)], p_bufs[0], sems_p[0])
        pltpu.async_copy(t_hbm.at[chunk_rows_slice(0)], t_bufs[0], sems_t[0])
        pltpu.async_copy(p_hbm.at[chunk_rows_slice(1)], p_bufs[1], sems_p[1])
        pltpu.async_copy(t_hbm.at[chunk_rows_slice(1)], t_bufs[1], sems_t[1])

        def chunk_sums(pb, tb, carry):
            def row_body(r, rcarry):
                a0, a1, an = rcarry
                for j in range(vecs_per_row):
                    p = pb[r, pl.ds(16 * j, 16)]
                    t = tb[r, pl.ds(16 * j, 16)]
                    e = p - t
                    s = jnp.minimum(jnp.maximum(e * e, _CLIP_LO), _CLIP_HI)
                    m0f = jnp.where(t > _EPS, 1.0, 0.0)
                    s0c = s * m0f
                    a0 = a0 + s0c
                    a1 = a1 + (s - s0c)
                    an = an + m0f
                return a0, a1, an

            return lax.fori_loop(0, _SC_CHUNK_ROWS, row_body, carry)

        def pair_body(g, carry):
            for b in range(2):
                idx = 2 * g + b
                pltpu.make_async_copy(
                    p_hbm.at[chunk_rows_slice(idx)], p_bufs[b], sems_p[b]).wait()
                pltpu.make_async_copy(
                    t_hbm.at[chunk_rows_slice(idx)], t_bufs[b], sems_t[b]).wait()
                carry = chunk_sums(p_bufs[b], t_bufs[b], carry)

                @pl.when(idx + 2 < n_chunks)
                def _prefetch():
                    pltpu.async_copy(
                        p_hbm.at[chunk_rows_slice(idx + 2)], p_bufs[b], sems_p[b])
                    pltpu.async_copy(
                        t_hbm.at[chunk_rows_slice(idx + 2)], t_bufs[b], sems_t[b])
            return carry

        zeros = jnp.zeros((16,), jnp.float32)
        a0, a1, an = lax.fori_loop(0, n_chunks // 2, pair_body,
                                   (zeros, zeros, zeros))

        res_v[pl.ds(0, 16)] = a0
        res_v[pl.ds(16, 16)] = a1
        res_v[pl.ds(32, 16)] = an
        pltpu.sync_copy(res_v, out_hbm.at[wid])

    return _sc_loss


def kernel(pred, truth):
    n_total = pred.size
    p2 = pred.reshape(-1, pred.shape[-1])
    t2 = truth.reshape(-1, truth.shape[-1])
    rows, cols = p2.shape

    rows_tc = rows - _SC_ROWS
    rows_per_worker = _SC_ROWS // _SC_WORKERS

    sc_parts = _make_sc_kernel(cols, rows_tc, rows_per_worker)(p2, t2)
    s0_tc, s1_tc, n0_tc = _tc_partial_sums(p2, t2, rows_tc)

    sc_parts = sc_parts.reshape(_SC_WORKERS, 3, 16)
    s0 = s0_tc[0, 0, 0] + jnp.sum(sc_parts[:, 0, :])
    s1 = s1_tc[0, 0, 0] + jnp.sum(sc_parts[:, 1, :])
    n0_sc = jnp.sum(sc_parts[:, 2, :].astype(jnp.int32))
    n0 = (n0_tc[0, 0, 0] + n0_sc).astype(jnp.float32)
    n1 = jnp.float32(n_total) - n0
    mean1 = s1 / jnp.maximum(n1, 1.0)
    mean0 = jnp.where(n0 > 0, s0 / jnp.maximum(n0, 1.0), mean1)
    return mean0 + mean1


# speedup vs baseline: 1.1899x; 1.1895x over previous
import jax
import jax.numpy as jnp
from jax import lax
from jax.experimental import pallas as pl
from jax.experimental.pallas import tpu as pltpu

_EPS = 0.001
_CLIP_LO = 1e-07
_CLIP_HI = 10000000.0

_ROWS_PER_BLOCK = 9216
_CHUNK_ROWS = 64


def _make_loss_kernel(n_steps, rows_per_block, chunk_rows):
    n_chunks = rows_per_block // chunk_rows

    def _loss_block_kernel(p_ref, t_ref, s0_ref, s1_ref, n0_ref,
                           acc0_ref, acc1_ref, accn_ref):
        @pl.when(pl.program_id(0) == 0)
        def _init():
            acc0_ref[...] = jnp.zeros_like(acc0_ref)
            acc1_ref[...] = jnp.zeros_like(acc1_ref)
            accn_ref[...] = jnp.zeros_like(accn_ref)

        w = p_ref.shape[-1]
        sub = chunk_rows // 8

        def body(i, carry):
            a0, a1, an = carry
            off = i * chunk_rows
            p = p_ref[pl.ds(off, chunk_rows), :]
            t = t_ref[pl.ds(off, chunk_rows), :]
            e = p - t
            s = jnp.minimum(jnp.maximum(e * e, _CLIP_LO), _CLIP_HI)
            m0f = jnp.where(t > _EPS, 1.0, 0.0)
            s0c = s * m0f
            a0 = a0 + jnp.sum(s0c.reshape(sub, 8, w), axis=0)
            a1 = a1 + jnp.sum(s.reshape(sub, 8, w), axis=0)
            an = an + jnp.sum(m0f.reshape(sub, 8, w), axis=0)
            return a0, a1, an

        zeros = jnp.zeros((8, w), jnp.float32)
        a0, a1, an = lax.fori_loop(0, n_chunks, body, (zeros, zeros, zeros))
        acc0_ref[...] += a0
        acc1_ref[...] += a1
        accn_ref[...] += an

        @pl.when(pl.program_id(0) == n_steps - 1)
        def _finish():
            s0_ref[...] = jnp.sum(acc0_ref[...]).reshape(1, 1, 1)
            s1_ref[...] = jnp.sum(acc1_ref[...] - acc0_ref[...]).reshape(1, 1, 1)
            n0_ref[...] = jnp.sum(accn_ref[...].astype(jnp.int32)).reshape(1, 1, 1)

    return _loss_block_kernel


def kernel(pred, truth):
    n_total = pred.size
    p2 = pred.reshape(-1, pred.shape[-1])
    t2 = truth.reshape(-1, truth.shape[-1])
    rows, cols = p2.shape
    n_steps = rows // _ROWS_PER_BLOCK

    in_spec = pl.BlockSpec((_ROWS_PER_BLOCK, cols), lambda i: (i, 0))
    out_spec = pl.BlockSpec((1, 1, 1), lambda i: (0, 0, 0))

    s0, s1, n0 = pl.pallas_call(
        _make_loss_kernel(n_steps, _ROWS_PER_BLOCK, _CHUNK_ROWS),
        grid=(n_steps,),
        in_specs=[in_spec, in_spec],
        out_specs=[out_spec, out_spec, out_spec],
        out_shape=[
            jax.ShapeDtypeStruct((1, 1, 1), jnp.float32),
            jax.ShapeDtypeStruct((1, 1, 1), jnp.float32),
            jax.ShapeDtypeStruct((1, 1, 1), jnp.int32),
        ],
        scratch_shapes=[
            pltpu.VMEM((8, cols), jnp.float32),
            pltpu.VMEM((8, cols), jnp.float32),
            pltpu.VMEM((8, cols), jnp.float32),
        ],
        compiler_params=pltpu.CompilerParams(
            dimension_semantics=("arbitrary",),
        ),
    )(p2, t2)

    s0 = s0[0, 0, 0]
    s1 = s1[0, 0, 0]
    n0 = n0[0, 0, 0].astype(jnp.float32)
    n1 = jnp.float32(n_total) - n0
    mean1 = s1 / jnp.maximum(n1, 1.0)
    mean0 = jnp.where(n0 > 0, s0 / jnp.maximum(n0, 1.0), mean1)
    return mean0 + mean1


# speedup vs baseline: 1.1922x; 1.0020x over previous
import jax
import jax.numpy as jnp
from jax import lax
from jax.experimental import pallas as pl
from jax.experimental.pallas import tpu as pltpu

_EPS = 0.001
_CLIP_LO = 1e-07
_CLIP_HI = 10000000.0

_ROWS_PER_BLOCK = 9216
_CHUNK_ROWS = 96


def _make_loss_kernel(n_steps, rows_per_block, chunk_rows):
    n_chunks = rows_per_block // chunk_rows

    def _loss_block_kernel(p_ref, t_ref, s0_ref, s1_ref, n0_ref,
                           acc0_ref, acc1_ref, accn_ref):
        @pl.when(pl.program_id(0) == 0)
        def _init():
            acc0_ref[...] = jnp.zeros_like(acc0_ref)
            acc1_ref[...] = jnp.zeros_like(acc1_ref)
            accn_ref[...] = jnp.zeros_like(accn_ref)

        w = p_ref.shape[-1]
        sub = chunk_rows // 8

        def body(i, carry):
            a0, a1, an = carry
            off = i * chunk_rows
            p = p_ref[pl.ds(off, chunk_rows), :]
            t = t_ref[pl.ds(off, chunk_rows), :]
            e = p - t
            s = jnp.minimum(jnp.maximum(e * e, _CLIP_LO), _CLIP_HI)
            m0f = jnp.where(t > _EPS, 1.0, 0.0)
            s0c = s * m0f
            a0 = a0 + jnp.sum(s0c.reshape(sub, 8, w), axis=0)
            a1 = a1 + jnp.sum(s.reshape(sub, 8, w), axis=0)
            an = an + jnp.sum(m0f.reshape(sub, 8, w), axis=0)
            return a0, a1, an

        zeros = jnp.zeros((8, w), jnp.float32)
        a0, a1, an = lax.fori_loop(0, n_chunks, body, (zeros, zeros, zeros))
        acc0_ref[...] += a0
        acc1_ref[...] += a1
        accn_ref[...] += an

        @pl.when(pl.program_id(0) == n_steps - 1)
        def _finish():
            s0_ref[...] = jnp.sum(acc0_ref[...]).reshape(1, 1, 1)
            s1_ref[...] = jnp.sum(acc1_ref[...] - acc0_ref[...]).reshape(1, 1, 1)
            n0_ref[...] = jnp.sum(accn_ref[...].astype(jnp.int32)).reshape(1, 1, 1)

    return _loss_block_kernel


def kernel(pred, truth):
    n_total = pred.size
    p2 = pred.reshape(-1, pred.shape[-1])
    t2 = truth.reshape(-1, truth.shape[-1])
    rows, cols = p2.shape
    n_steps = rows // _ROWS_PER_BLOCK

    in_spec = pl.BlockSpec((_ROWS_PER_BLOCK, cols), lambda i: (i, 0))
    out_spec = pl.BlockSpec((1, 1, 1), lambda i: (0, 0, 0))

    s0, s1, n0 = pl.pallas_call(
        _make_loss_kernel(n_steps, _ROWS_PER_BLOCK, _CHUNK_ROWS),
        grid=(n_steps,),
        in_specs=[in_spec, in_spec],
        out_specs=[out_spec, out_spec, out_spec],
        out_shape=[
            jax.ShapeDtypeStruct((1, 1, 1), jnp.float32),
            jax.ShapeDtypeStruct((1, 1, 1), jnp.float32),
            jax.ShapeDtypeStruct((1, 1, 1), jnp.int32),
        ],
        scratch_shapes=[
            pltpu.VMEM((8, cols), jnp.float32),
            pltpu.VMEM((8, cols), jnp.float32),
            pltpu.VMEM((8, cols), jnp.float32),
        ],
        compiler_params=pltpu.CompilerParams(
            dimension_semantics=("arbitrary",),
        ),
    )(p2, t2)

    s0 = s0[0, 0, 0]
    s1 = s1[0, 0, 0]
    n0 = n0[0, 0, 0].astype(jnp.float32)
    n1 = jnp.float32(n_total) - n0
    mean1 = s1 / jnp.maximum(n1, 1.0)
    mean0 = jnp.where(n0 > 0, s0 / jnp.maximum(n0, 1.0), mean1)
    return mean0 + mean1
